# baseline (speedup 1.0000x reference)
import functools

import jax
import jax.numpy as jnp
from jax import lax
from jax.experimental import pallas as pl
from jax.experimental.pallas import tpu as pltpu
from jax.experimental.pallas import tpu_sc as plsc

_N1, _N2 = 4000, 1000
_D_IN, _D_HID, _D_OUT = 256, 512, 256

_NC, _NS = 2, 16
_NW = _NC * _NS
_K = 128
_G = 128


def _make_agg(R, D, CH):
    G = D // _G
    stripe = R // _NS
    mesh = plsc.VectorSubcoreMesh(core_axis_name="c", subcore_axis_name="s")

    @functools.partial(
        pl.kernel,
        mesh=mesh,
        out_type=jax.ShapeDtypeStruct((_NC, G, R, _G), jnp.float32),
        scratch_types=[
            pltpu.VMEM((2, _K), jnp.int32),
            pltpu.VMEM((_K,), jnp.int32),
            pltpu.VMEM((2, _K), jnp.int32),
            pltpu.VMEM((2, _K, _G), jnp.float32),
            pltpu.VMEM_SHARED((G, R, _G), jnp.float32),
            pltpu.SemaphoreType.DMA,
            pltpu.SemaphoreType.DMA,
        ],
    )
    def agg(feat_hbm, src_hbm, dst_hbm, zrow_hbm,
            out_hbm, didx, sidx, gidx, rows, acc, sem0, sem1):
        cid = lax.axis_index("c")
        sid = lax.axis_index("s")
        w = cid * _NS + sid
        row0 = sid * stripe
        sems = (sem0, sem1)
        for g in range(G):
            pltpu.sync_copy(zrow_hbm.at[pl.ds(0, stripe)],
                            acc.at[g, pl.ds(row0, stripe)])
        plsc.subcore_barrier()

        def load_chunk(c, cpar):
            base = (w * CH + c) * _K
            pltpu.sync_copy(src_hbm.at[pl.ds(base, _K)], sidx)
            pltpu.sync_copy(dst_hbm.at[pl.ds(base, _K)], didx.at[cpar])

        def start_gather(g, spar):
            for j in range(_K // 16):
                sl = pl.ds(j * 16, 16)
                gidx[spar, sl] = sidx[sl] * G + g
            pltpu.async_copy(feat_hbm.at[gidx.at[spar]], rows.at[spar],
                             sems[spar])

        def wait_gather(spar):
            pltpu.make_async_copy(zrow_hbm.at[pl.ds(0, _K)], rows.at[spar],
                                  sems[spar]).wait()

        load_chunk(0, 0)
        start_gather(0, 0)

        def body(i, carry):
            for k in range(2 * G):
                cpar, g, spar = k // G, k % G, k % 2
                nk = k + 1
                if nk % G == 0:
                    load_chunk(2 * i + nk // G, (nk // G) % 2)
                start_gather(nk % G, nk % 2)
                wait_gather(spar)
                pltpu.sync_copy(rows.at[spar], acc.at[g].at[didx.at[cpar]],
                                add=True)
            return carry

        lax.fori_loop(0, CH // 2, body, 0)
        wait_gather(0)
        plsc.subcore_barrier()
        for g in range(G):
            pltpu.sync_copy(acc.at[g, pl.ds(row0, stripe)],
                            out_hbm.at[cid, g, pl.ds(row0, stripe)])

    return agg


def _make_counts(R1, CH1, R2, CH2):
    s1, s2 = R1 // _NS, R2 // _NS
    mesh = plsc.VectorSubcoreMesh(core_axis_name="c", subcore_axis_name="s")

    @functools.partial(
        pl.kernel,
        mesh=mesh,
        out_type=(
            jax.ShapeDtypeStruct((_NC, R1, _G), jnp.float32),
            jax.ShapeDtypeStruct((_NC, R2, _G), jnp.float32),
        ),
        scratch_types=[
            pltpu.VMEM((_K,), jnp.int32),
            pltpu.VMEM((_K, _G), jnp.float32),
            pltpu.VMEM_SHARED((R1, _G), jnp.float32),
            pltpu.VMEM_SHARED((R2, _G), jnp.float32),
        ],
    )
    def cntk(dst1_hbm, dst2_hbm, zrow_hbm, ones_hbm,
             cnt1_hbm, cnt2_hbm, didx, ones, acc1, acc2):
        cid = lax.axis_index("c")
        sid = lax.axis_index("s")
        w = cid * _NS + sid
        pltpu.sync_copy(zrow_hbm.at[pl.ds(0, s1)], acc1.at[pl.ds(sid * s1, s1)])
        pltpu.sync_copy(zrow_hbm.at[pl.ds(0, s2)], acc2.at[pl.ds(sid * s2, s2)])
        pltpu.sync_copy(ones_hbm, ones)
        plsc.subcore_barrier()

        def body1(c, carry):
            pltpu.sync_copy(dst1_hbm.at[pl.ds((w * CH1 + c) * _K, _K)], didx)
            pltpu.sync_copy(ones, acc1.at[didx], add=True)
            return carry

        def body2(c, carry):
            pltpu.sync_copy(dst2_hbm.at[pl.ds((w * CH2 + c) * _K, _K)], didx)
            pltpu.sync_copy(ones, acc2.at[didx], add=True)
            return carry

        lax.fori_loop(0, CH1, body1, 0)
        lax.fori_loop(0, CH2, body2, 0)
        plsc.subcore_barrier()
        pltpu.sync_copy(acc1.at[pl.ds(sid * s1, s1)],
                        cnt1_hbm.at[cid, pl.ds(sid * s1, s1)])
        pltpu.sync_copy(acc2.at[pl.ds(sid * s2, s2)],
                        cnt2_hbm.at[cid, pl.ds(sid * s2, s2)])

    return cntk


def _mean_linear_relu(parts, cnts, W, b, bm):
    G, R = parts.shape[1], parts.shape[2]
    Dout = W.shape[1]

    def spec(c, g):
        return pl.BlockSpec((1, 1, bm, _G), lambda i, c=c, g=g: (c, g, i, 0))

    def body(*refs):
        p_refs = refs[: _NC * G]
        c_refs = refs[_NC * G: _NC * G + _NC]
        w_ref, b_ref, o_ref = refs[_NC * G + _NC:]
        cnt = sum(c[0, :, 0:1] for c in c_refs)
        inv = 1.0 / jnp.maximum(cnt, 1.0)
        acc = jnp.zeros((bm, Dout), jnp.float32)
        for g in range(G):
            p = p_refs[g][0, 0]
            for c in range(1, _NC):
                p = p + p_refs[c * G + g][0, 0]
            acc = acc + jnp.dot(p * inv, w_ref[pl.ds(g * _G, _G), :],
                                preferred_element_type=jnp.float32)
        o_ref[...] = jax.nn.relu(acc + b_ref[...])

    args = [parts] * (_NC * G) + [cnts] * _NC + [W, b.reshape(1, Dout)]
    in_specs = ([spec(c, g) for c in range(_NC) for g in range(G)]
                + [pl.BlockSpec((1, bm, _G), lambda i, c=c: (c, i, 0))
                   for c in range(_NC)]
                + [pl.BlockSpec(W.shape, lambda i: (0, 0)),
                   pl.BlockSpec((1, Dout), lambda i: (0, 0))])
    return pl.pallas_call(
        body,
        grid=(R // bm,),
        in_specs=in_specs,
        out_specs=pl.BlockSpec((bm, Dout), lambda i: (i, 0)),
        out_shape=jax.ShapeDtypeStruct((R, Dout), jnp.float32),
    )(*args)


def _pad_edges(src, dst, e_pad, n_src, dummy_lo, dummy_hi):
    pad = e_pad + _K - src.shape[0]
    i = jnp.arange(pad, dtype=jnp.int32)
    s = jnp.concatenate([src.astype(jnp.int32), i % n_src])
    d = jnp.concatenate([dst.astype(jnp.int32),
                         dummy_lo + i % (dummy_hi - dummy_lo)])
    return s, d, e_pad // (_NW * _K)


def kernel(x, src0, dst0, src1, dst1, W1, b1, W2, b2):
    R1, R2 = 4096, 1024
    ones = jnp.ones((_K, _G), jnp.float32)
    zrow = jnp.zeros((R1 // _NS, _G), jnp.float32)

    s0, d0, ch0 = _pad_edges(src0, dst0, 65536, 10000, _N1, R1)
    s1, d1, ch1 = _pad_edges(src1, dst1, 16384, R1, _N2, R2)

    C1, C2 = _make_counts(R1, ch0, R2, ch1)(d0, d1, zrow, ones)

    P1 = _make_agg(R1, _D_IN, ch0)(x.reshape(-1, _G), s0, d0, zrow)
    h1 = _mean_linear_relu(P1, C1, W1, b1, bm=256)

    P2 = _make_agg(R2, _D_HID, ch1)(h1.reshape(-1, _G), s1, d1, zrow)
    h2 = _mean_linear_relu(P2, C2, W2, b2, bm=256)
    return h2[:_N2]


# speedup vs baseline: 1.0980x; 1.0980x over previous
import functools

import jax
import jax.numpy as jnp
from jax import lax
from jax.experimental import pallas as pl
from jax.experimental.pallas import tpu as pltpu
from jax.experimental.pallas import tpu_sc as plsc

_N1, _N2 = 4000, 1000
_D_IN, _D_HID, _D_OUT = 256, 512, 256

_NC, _NS = 2, 16
_NW = _NC * _NS
_K = 128
_G = 128


def _make_agg(R, D, CH):
    G = D // _G
    stripe = R // _NS
    mesh = plsc.VectorSubcoreMesh(core_axis_name="c", subcore_axis_name="s")

    @functools.partial(
        pl.kernel,
        mesh=mesh,
        out_type=jax.ShapeDtypeStruct((_NC, G, R, _G), jnp.float32),
        scratch_types=[
            pltpu.VMEM((2, _K), jnp.int32),
            pltpu.VMEM((_K,), jnp.int32),
            pltpu.VMEM((2, _K), jnp.int32),
            pltpu.VMEM((2, _K, _G), jnp.float32),
            pltpu.VMEM_SHARED((G, R, _G), jnp.float32),
            pltpu.SemaphoreType.DMA,
            pltpu.SemaphoreType.DMA,
            pltpu.SemaphoreType.DMA,
            pltpu.SemaphoreType.DMA,
        ],
    )
    def agg(feat_hbm, src_hbm, dst_hbm, zrow_hbm,
            out_hbm, didx, sidx, gidx, rows, acc, sem0, sem1, sem2, sem3):
        cid = lax.axis_index("c")
        sid = lax.axis_index("s")
        w = cid * _NS + sid
        row0 = sid * stripe
        semg = (sem0, sem1)
        sems = (sem2, sem3)
        for g in range(G):
            pltpu.sync_copy(zrow_hbm.at[pl.ds(0, stripe)],
                            acc.at[g, pl.ds(row0, stripe)])
        plsc.subcore_barrier()

        def load_chunk(c, cpar):
            base = (w * CH + c) * _K
            pltpu.sync_copy(src_hbm.at[pl.ds(base, _K)], sidx)
            pltpu.sync_copy(dst_hbm.at[pl.ds(base, _K)], didx.at[cpar])

        def start_gather(g, spar):
            for j in range(_K // 16):
                sl = pl.ds(j * 16, 16)
                gidx[spar, sl] = sidx[sl] * G + g
            pltpu.async_copy(feat_hbm.at[gidx.at[spar]], rows.at[spar],
                             semg[spar])

        def wait_dma(spar, sem):
            pltpu.make_async_copy(zrow_hbm.at[pl.ds(0, _K)], rows.at[spar],
                                  sem).wait()

        load_chunk(0, 0)
        start_gather(0, 0)

        def body(i, carry):
            for k in range(2 * G):
                cpar, g, spar = k // G, k % G, k % 2
                nk = k + 1
                if nk % G == 0:
                    load_chunk(2 * i + nk // G, (nk // G) % 2)
                if k == 0:
                    @pl.when(i > 0)
                    def _():
                        wait_dma(1 - spar, sems[1 - spar])
                else:
                    wait_dma(1 - spar, sems[1 - spar])
                start_gather(nk % G, 1 - spar)
                wait_dma(spar, semg[spar])
                pltpu.async_copy(rows.at[spar], acc.at[g].at[didx.at[cpar]],
                                 sems[spar], add=True)
            return carry

        lax.fori_loop(0, CH // 2, body, 0)
        wait_dma(1, sems[1])
        wait_dma(0, semg[0])
        plsc.subcore_barrier()
        for g in range(G):
            pltpu.sync_copy(acc.at[g, pl.ds(row0, stripe)],
                            out_hbm.at[cid, g, pl.ds(row0, stripe)])

    return agg


def _make_counts(R1, CH1, R2, CH2):
    s1, s2 = R1 // _NS, R2 // _NS
    mesh = plsc.VectorSubcoreMesh(core_axis_name="c", subcore_axis_name="s")

    @functools.partial(
        pl.kernel,
        mesh=mesh,
        out_type=(
            jax.ShapeDtypeStruct((_NC, R1, _G), jnp.float32),
            jax.ShapeDtypeStruct((_NC, R2, _G), jnp.float32),
        ),
        scratch_types=[
            pltpu.VMEM((2, _K), jnp.int32),
            pltpu.VMEM((_K, _G), jnp.float32),
            pltpu.VMEM_SHARED((R1, _G), jnp.float32),
            pltpu.VMEM_SHARED((R2, _G), jnp.float32),
            pltpu.SemaphoreType.DMA,
            pltpu.SemaphoreType.DMA,
        ],
    )
    def cntk(dst1_hbm, dst2_hbm, zrow_hbm, ones_hbm,
             cnt1_hbm, cnt2_hbm, didx, ones, acc1, acc2, sem0, sem1):
        cid = lax.axis_index("c")
        sid = lax.axis_index("s")
        w = cid * _NS + sid
        pltpu.sync_copy(zrow_hbm.at[pl.ds(0, s1)], acc1.at[pl.ds(sid * s1, s1)])
        pltpu.sync_copy(zrow_hbm.at[pl.ds(0, s2)], acc2.at[pl.ds(sid * s2, s2)])
        pltpu.sync_copy(ones_hbm, ones)
        plsc.subcore_barrier()

        sems = (sem0, sem1)

        def wait_scat(p):
            pltpu.make_async_copy(zrow_hbm.at[pl.ds(0, _K)], ones,
                                  sems[p]).wait()

        def body1(i, carry):
            for p in range(2):
                @pl.when(i > 0)
                def _():
                    wait_scat(p)
                c = 2 * i + p
                pltpu.sync_copy(dst1_hbm.at[pl.ds((w * CH1 + c) * _K, _K)],
                                didx.at[p])
                pltpu.async_copy(ones, acc1.at[didx.at[p]], sems[p], add=True)
            return carry

        def body2(i, carry):
            for p in range(2):
                wait_scat(p)
                c = 2 * i + p
                pltpu.sync_copy(dst2_hbm.at[pl.ds((w * CH2 + c) * _K, _K)],
                                didx.at[p])
                pltpu.async_copy(ones, acc2.at[didx.at[p]], sems[p], add=True)
            return carry

        lax.fori_loop(0, CH1 // 2, body1, 0)
        lax.fori_loop(0, CH2 // 2, body2, 0)
        for p in range(2):
            wait_scat(p)
        plsc.subcore_barrier()
        pltpu.sync_copy(acc1.at[pl.ds(sid * s1, s1)],
                        cnt1_hbm.at[cid, pl.ds(sid * s1, s1)])
        pltpu.sync_copy(acc2.at[pl.ds(sid * s2, s2)],
                        cnt2_hbm.at[cid, pl.ds(sid * s2, s2)])

    return cntk


def _mean_linear_relu(parts, cnts, W, b, bm):
    G, R = parts.shape[1], parts.shape[2]
    Dout = W.shape[1]

    def spec(c, g):
        return pl.BlockSpec((1, 1, bm, _G), lambda i, c=c, g=g: (c, g, i, 0))

    def body(*refs):
        p_refs = refs[: _NC * G]
        c_refs = refs[_NC * G: _NC * G + _NC]
        w_ref, b_ref, o_ref = refs[_NC * G + _NC:]
        cnt = sum(c[0, :, 0:1] for c in c_refs)
        inv = 1.0 / jnp.maximum(cnt, 1.0)
        acc = jnp.zeros((bm, Dout), jnp.float32)
        for g in range(G):
            p = p_refs[g][0, 0]
            for c in range(1, _NC):
                p = p + p_refs[c * G + g][0, 0]
            acc = acc + jnp.dot(p * inv, w_ref[pl.ds(g * _G, _G), :],
                                preferred_element_type=jnp.float32)
        o_ref[...] = jax.nn.relu(acc + b_ref[...])

    args = [parts] * (_NC * G) + [cnts] * _NC + [W, b.reshape(1, Dout)]
    in_specs = ([spec(c, g) for c in range(_NC) for g in range(G)]
                + [pl.BlockSpec((1, bm, _G), lambda i, c=c: (c, i, 0))
                   for c in range(_NC)]
                + [pl.BlockSpec(W.shape, lambda i: (0, 0)),
                   pl.BlockSpec((1, Dout), lambda i: (0, 0))])
    return pl.pallas_call(
        body,
        grid=(R // bm,),
        in_specs=in_specs,
        out_specs=pl.BlockSpec((bm, Dout), lambda i: (i, 0)),
        out_shape=jax.ShapeDtypeStruct((R, Dout), jnp.float32),
    )(*args)


def _pad_edges(src, dst, e_pad, n_src, dummy_lo, dummy_hi):
    pad = e_pad + _K - src.shape[0]
    i = jnp.arange(pad, dtype=jnp.int32)
    s = jnp.concatenate([src.astype(jnp.int32), i % n_src])
    d = jnp.concatenate([dst.astype(jnp.int32),
                         dummy_lo + i % (dummy_hi - dummy_lo)])
    return s, d, e_pad // (_NW * _K)


def kernel(x, src0, dst0, src1, dst1, W1, b1, W2, b2):
    R1, R2 = 4096, 1024
    ones = jnp.ones((_K, _G), jnp.float32)
    zrow = jnp.zeros((R1 // _NS, _G), jnp.float32)

    s0, d0, ch0 = _pad_edges(src0, dst0, 65536, 10000, _N1, R1)
    s1, d1, ch1 = _pad_edges(src1, dst1, 16384, R1, _N2, R2)

    C1, C2 = _make_counts(R1, ch0, R2, ch1)(d0, d1, zrow, ones)

    P1 = _make_agg(R1, _D_IN, ch0)(x.reshape(-1, _G), s0, d0, zrow)
    h1 = _mean_linear_relu(P1, C1, W1, b1, bm=256)

    P2 = _make_agg(R2, _D_HID, ch1)(h1.reshape(-1, _G), s1, d1, zrow)
    h2 = _mean_linear_relu(P2, C2, W2, b2, bm=256)
    return h2[:_N2]


# speedup vs baseline: 1.2314x; 1.1215x over previous
import functools

import jax
import jax.numpy as jnp
from jax import lax
from jax.experimental import pallas as pl
from jax.experimental.pallas import tpu as pltpu
from jax.experimental.pallas import tpu_sc as plsc

_N1, _N2 = 4000, 1000
_D_IN, _D_HID, _D_OUT = 256, 512, 256

_NC, _NS = 2, 16
_NW = _NC * _NS
_K = 128
_G = 128


def _make_agg(R, D, CH):
    G = D // _G
    stripe = R // _NS
    mesh = plsc.VectorSubcoreMesh(core_axis_name="c", subcore_axis_name="s")

    @functools.partial(
        pl.kernel,
        mesh=mesh,
        out_type=jax.ShapeDtypeStruct((_NC, G, R, _G), jnp.float32),
        scratch_types=[
            pltpu.VMEM((2, _K), jnp.int32),
            pltpu.VMEM((_K,), jnp.int32),
            pltpu.VMEM((2, _K), jnp.int32),
            pltpu.VMEM((2, _K, _G), jnp.float32),
            pltpu.VMEM_SHARED((G, R, _G), jnp.float32),
            pltpu.SemaphoreType.DMA,
            pltpu.SemaphoreType.DMA,
            pltpu.SemaphoreType.DMA,
            pltpu.SemaphoreType.DMA,
        ],
    )
    def agg(feat_hbm, src_hbm, dst_hbm, zrow_hbm,
            out_hbm, didx, sidx, gidx, rows, acc, sem0, sem1, sem2, sem3):
        cid = lax.axis_index("c")
        sid = lax.axis_index("s")
        w = cid * _NS + sid
        row0 = sid * stripe
        semg = (sem0, sem1)
        sems = (sem2, sem3)
        for g in range(G):
            pltpu.sync_copy(zrow_hbm.at[pl.ds(0, stripe)],
                            acc.at[g, pl.ds(row0, stripe)])
        plsc.subcore_barrier()

        def load_chunk(c, cpar):
            base = (w * CH + c) * _K
            pltpu.sync_copy(src_hbm.at[pl.ds(base, _K)], sidx)
            pltpu.sync_copy(dst_hbm.at[pl.ds(base, _K)], didx.at[cpar])

        def start_gather(g, spar):
            for j in range(_K // 16):
                sl = pl.ds(j * 16, 16)
                gidx[spar, sl] = sidx[sl] * G + g
            pltpu.async_copy(feat_hbm.at[gidx.at[spar]], rows.at[spar],
                             semg[spar])

        def wait_dma(spar, sem):
            pltpu.make_async_copy(zrow_hbm.at[pl.ds(0, _K)], rows.at[spar],
                                  sem).wait()

        load_chunk(0, 0)
        start_gather(0, 0)

        def body(i, carry):
            for k in range(2 * G):
                cpar, g, spar = k // G, k % G, k % 2
                nk = k + 1
                if nk % G == 0:
                    load_chunk(2 * i + nk // G, (nk // G) % 2)
                if k == 0:
                    @pl.when(i > 0)
                    def _():
                        wait_dma(1 - spar, sems[1 - spar])
                else:
                    wait_dma(1 - spar, sems[1 - spar])
                start_gather(nk % G, 1 - spar)
                wait_dma(spar, semg[spar])
                pltpu.async_copy(rows.at[spar], acc.at[g].at[didx.at[cpar]],
                                 sems[spar], add=True)
            return carry

        lax.fori_loop(0, CH // 2, body, 0)
        wait_dma(1, sems[1])
        wait_dma(0, semg[0])
        plsc.subcore_barrier()
        for g in range(G):
            pltpu.sync_copy(acc.at[g, pl.ds(row0, stripe)],
                            out_hbm.at[cid, g, pl.ds(row0, stripe)])

    return agg


def _make_counts(R1, CH1, R2, CH2):
    s1, s2 = R1 // _NS, R2 // _NS
    mesh = plsc.VectorSubcoreMesh(core_axis_name="c", subcore_axis_name="s")

    @functools.partial(
        pl.kernel,
        mesh=mesh,
        out_type=(
            jax.ShapeDtypeStruct((_NC, R1, _G), jnp.float32),
            jax.ShapeDtypeStruct((_NC, R2, _G), jnp.float32),
        ),
        scratch_types=[
            pltpu.VMEM((2, _K), jnp.int32),
            pltpu.VMEM((_K, _G), jnp.float32),
            pltpu.VMEM_SHARED((R1, _G), jnp.float32),
            pltpu.VMEM_SHARED((R2, _G), jnp.float32),
            pltpu.SemaphoreType.DMA,
            pltpu.SemaphoreType.DMA,
        ],
    )
    def cntk(dst1_hbm, dst2_hbm, zrow_hbm, ones_hbm,
             cnt1_hbm, cnt2_hbm, didx, ones, acc1, acc2, sem0, sem1):
        cid = lax.axis_index("c")
        sid = lax.axis_index("s")
        w = cid * _NS + sid
        pltpu.sync_copy(zrow_hbm.at[pl.ds(0, s1)], acc1.at[pl.ds(sid * s1, s1)])
        pltpu.sync_copy(zrow_hbm.at[pl.ds(0, s2)], acc2.at[pl.ds(sid * s2, s2)])
        pltpu.sync_copy(ones_hbm, ones)
        plsc.subcore_barrier()

        sems = (sem0, sem1)

        def wait_scat(p):
            pltpu.make_async_copy(zrow_hbm.at[pl.ds(0, _K)], ones,
                                  sems[p]).wait()

        def body1(i, carry):
            for p in range(2):
                @pl.when(i > 0)
                def _():
                    wait_scat(p)
                c = 2 * i + p
                pltpu.sync_copy(dst1_hbm.at[pl.ds((w * CH1 + c) * _K, _K)],
                                didx.at[p])
                pltpu.async_copy(ones, acc1.at[didx.at[p]], sems[p], add=True)
            return carry

        def body2(i, carry):
            for p in range(2):
                wait_scat(p)
                c = 2 * i + p
                pltpu.sync_copy(dst2_hbm.at[pl.ds((w * CH2 + c) * _K, _K)],
                                didx.at[p])
                pltpu.async_copy(ones, acc2.at[didx.at[p]], sems[p], add=True)
            return carry

        lax.fori_loop(0, CH1 // 2, body1, 0)
        lax.fori_loop(0, CH2 // 2, body2, 0)
        for p in range(2):
            wait_scat(p)
        plsc.subcore_barrier()
        pltpu.sync_copy(acc1.at[pl.ds(sid * s1, s1)],
                        cnt1_hbm.at[cid, pl.ds(sid * s1, s1)])
        pltpu.sync_copy(acc2.at[pl.ds(sid * s2, s2)],
                        cnt2_hbm.at[cid, pl.ds(sid * s2, s2)])

    return cntk


def _mean_linear_relu(parts, cnts, W, b, W2, bm):
    G, R = parts.shape[1], parts.shape[2]
    Dout = W.shape[1]
    Dout2 = W2.shape[1]

    def spec(c, g):
        return pl.BlockSpec((1, 1, bm, _G), lambda i, c=c, g=g: (c, g, i, 0))

    def body(*refs):
        p_refs = refs[: _NC * G]
        c_refs = refs[_NC * G: _NC * G + _NC]
        w_ref, b_ref, w2_ref, o2_ref = refs[_NC * G + _NC:]
        cnt = sum(c[0, :, 0:1] for c in c_refs)
        inv = 1.0 / jnp.maximum(cnt, 1.0)
        acc = jnp.zeros((bm, Dout), jnp.float32)
        for g in range(G):
            p = p_refs[g][0, 0]
            for c in range(1, _NC):
                p = p + p_refs[c * G + g][0, 0]
            acc = acc + jnp.dot(p * inv, w_ref[pl.ds(g * _G, _G), :],
                                preferred_element_type=jnp.float32)
        h = jax.nn.relu(acc + b_ref[...])
        o2_ref[...] = jnp.dot(h, w2_ref[...],
                              preferred_element_type=jnp.float32)

    args = ([parts] * (_NC * G) + [cnts] * _NC
            + [W, b.reshape(1, Dout), W2])
    in_specs = ([spec(c, g) for c in range(_NC) for g in range(G)]
                + [pl.BlockSpec((1, bm, _G), lambda i, c=c: (c, i, 0))
                   for c in range(_NC)]
                + [pl.BlockSpec(W.shape, lambda i: (0, 0)),
                   pl.BlockSpec((1, Dout), lambda i: (0, 0)),
                   pl.BlockSpec(W2.shape, lambda i: (0, 0))])
    return pl.pallas_call(
        body,
        grid=(R // bm,),
        in_specs=in_specs,
        out_specs=pl.BlockSpec((bm, Dout2), lambda i: (i, 0)),
        out_shape=jax.ShapeDtypeStruct((R, Dout2), jnp.float32),
    )(*args)


def _mean_bias_relu(parts, cnts, b, bm):
    G, R = parts.shape[1], parts.shape[2]
    Dout = G * _G

    def spec(c, g):
        return pl.BlockSpec((1, 1, bm, _G), lambda i, c=c, g=g: (c, g, i, 0))

    def body(*refs):
        p_refs = refs[: _NC * G]
        c_refs = refs[_NC * G: _NC * G + _NC]
        b_ref, o_ref = refs[_NC * G + _NC:]
        cnt = sum(c[0, :, 0:1] for c in c_refs)
        inv = 1.0 / jnp.maximum(cnt, 1.0)
        cols = []
        for g in range(G):
            p = p_refs[g][0, 0]
            for c in range(1, _NC):
                p = p + p_refs[c * G + g][0, 0]
            cols.append(p * inv)
        o_ref[...] = jax.nn.relu(jnp.concatenate(cols, axis=1) + b_ref[...])

    args = [parts] * (_NC * G) + [cnts] * _NC + [b.reshape(1, Dout)]
    in_specs = ([spec(c, g) for c in range(_NC) for g in range(G)]
                + [pl.BlockSpec((1, bm, _G), lambda i, c=c: (c, i, 0))
                   for c in range(_NC)]
                + [pl.BlockSpec((1, Dout), lambda i: (0, 0))])
    return pl.pallas_call(
        body,
        grid=(R // bm,),
        in_specs=in_specs,
        out_specs=pl.BlockSpec((bm, Dout), lambda i: (i, 0)),
        out_shape=jax.ShapeDtypeStruct((R, Dout), jnp.float32),
    )(*args)


def _pad_edges(src, dst, e_pad, n_src, dummy_lo, dummy_hi):
    pad = e_pad + _K - src.shape[0]
    i = jnp.arange(pad, dtype=jnp.int32)
    s = jnp.concatenate([src.astype(jnp.int32), i % n_src])
    d = jnp.concatenate([dst.astype(jnp.int32),
                         dummy_lo + i % (dummy_hi - dummy_lo)])
    return s, d, e_pad // (_NW * _K)


def kernel(x, src0, dst0, src1, dst1, W1, b1, W2, b2):
    R1, R2 = 4096, 1024
    ones = jnp.ones((_K, _G), jnp.float32)
    zrow = jnp.zeros((R1 // _NS, _G), jnp.float32)

    s0, d0, ch0 = _pad_edges(src0, dst0, 65536, 10000, _N1, R1)
    s1, d1, ch1 = _pad_edges(src1, dst1, 16384, R1, _N2, R2)

    C1, C2 = _make_counts(R1, ch0, R2, ch1)(d0, d1, zrow, ones)

    P1 = _make_agg(R1, _D_IN, ch0)(x.reshape(-1, _G), s0, d0, zrow)
    y = _mean_linear_relu(P1, C1, W1, b1, W2, bm=256)

    P2 = _make_agg(R2, _D_OUT, ch1)(y.reshape(-1, _G), s1, d1, zrow)
    h2 = _mean_bias_relu(P2, C2, b2, bm=256)
    return h2[:_N2]
